# baseline (speedup 1.0000x reference)
import functools

import jax
import jax.numpy as jnp
from jax import lax
from jax.experimental import pallas as pl
from jax.experimental.pallas import tpu as pltpu
from jax.experimental.pallas import tpu_sc as plsc

V = 100000
EMB = 400
U = 16
G4 = 4 * U
FEAT = 16
B = 1024
L = 200
LB = B * L


_S1_ROWS = 1000


def _proj_body(emb_ref, we_ref, out_ref):
    out_ref[...] = jnp.dot(emb_ref[...], we_ref[...],
                           preferred_element_type=jnp.float32)


def _project_table(emb_table, w_e):
    return pl.pallas_call(
        _proj_body,
        grid=(V // _S1_ROWS,),
        in_specs=[
            pl.BlockSpec((_S1_ROWS, EMB), lambda i: (i, 0)),
            pl.BlockSpec((EMB, G4), lambda i: (0, 0)),
        ],
        out_specs=pl.BlockSpec((_S1_ROWS, G4), lambda i: (i, 0)),
        out_shape=jax.ShapeDtypeStruct((V, G4), jnp.float32),
    )(emb_table, w_e)


_NC, _NS = 2, 16
_NW = _NC * _NS
_ROWS_W = LB // _NW
_JCH = 128
_JPS = 10
_SUP = _ROWS_W // (_JPS * _JCH)


def _sc_gather_body(table_hbm, idx_hbm, out_hbm, idx_v, rows_v, sem):
    wid = lax.axis_index("s") * _NC + lax.axis_index("c")
    row0 = wid * _ROWS_W
    pltpu.sync_copy(idx_hbm.at[wid], idx_v)

    @pl.loop(0, _SUP)
    def _super(s):
        copies = []
        for j in range(_JPS):
            copies.append(pltpu.async_copy(
                table_hbm.at[idx_v.at[s * _JPS + j]],
                rows_v.at[pl.ds(j * _JCH, _JCH)], sem))
        for c in copies:
            c.wait()
        pltpu.sync_copy(
            rows_v, out_hbm.at[pl.ds(row0 + s * _JPS * _JCH, _JPS * _JCH)])


@functools.cache
def _make_sc_gather():
    return functools.partial(
        pl.kernel,
        out_type=jax.ShapeDtypeStruct((LB, G4), jnp.float32),
        mesh=plsc.VectorSubcoreMesh(core_axis_name="c", subcore_axis_name="s"),
        scratch_types=[
            pltpu.VMEM((_ROWS_W // _JCH, _JCH), jnp.int32),
            pltpu.VMEM((_JPS * _JCH, G4), jnp.float32),
            pltpu.SemaphoreType.DMA,
        ],
        compiler_params=pltpu.CompilerParams(use_tc_tiling_on_sc=False),
    )(_sc_gather_body)


def _sc_gather(table, idx3d):
    return _make_sc_gather()(table, idx3d)


_XCH = 8


def _x1p_body(x1_ref, wf_ref, bias_ref, out_ref):
    for l in range(_XCH):
        out_ref[l] = (jnp.dot(x1_ref[:, l, :], wf_ref[...],
                              preferred_element_type=jnp.float32)
                      + bias_ref[...])


def _x1_proj(x1, w_f, bias2d):
    return pl.pallas_call(
        _x1p_body,
        grid=(L // _XCH,),
        in_specs=[
            pl.BlockSpec((B, _XCH, FEAT), lambda i: (0, i, 0)),
            pl.BlockSpec((FEAT, G4), lambda i: (0, 0)),
            pl.BlockSpec((1, G4), lambda i: (0, 0)),
        ],
        out_specs=pl.BlockSpec((_XCH, B, G4), lambda i: (i, 0, 0)),
        out_shape=jax.ShapeDtypeStruct((L, B, G4), jnp.float32),
    )(x1, w_f, bias2d)


_TCH = 10


def _lstm_body(g_ref, zx0_ref, rec_ref, h_out_ref, h_s, c_s, zx_s):
    i = pl.program_id(0)

    @pl.when(i == 0)
    def _init():
        h_s[...] = jnp.zeros_like(h_s)
        c_s[...] = jnp.zeros_like(c_s)

    for tt in range(_TCH):
        zx_s[tt] = g_ref[tt] + zx0_ref[tt]

    def step(t, hc):
        h, c = hc
        z = zx_s[t] + jnp.dot(h, rec_ref[...],
                              preferred_element_type=jnp.float32)
        i_g = jax.nn.sigmoid(z[:, 0:U])
        f_g = jax.nn.sigmoid(z[:, U:2 * U])
        g_g = jnp.tanh(z[:, 2 * U:3 * U])
        o_g = jax.nn.sigmoid(z[:, 3 * U:4 * U])
        c_new = f_g * c + i_g * g_g
        h_new = o_g * jnp.tanh(c_new)
        return h_new, c_new

    h, c = lax.fori_loop(0, _TCH, step, (h_s[...], c_s[...]))
    h_s[...] = h
    c_s[...] = c
    h_out_ref[...] = h


def _lstm(g3, zx0, rec):
    return pl.pallas_call(
        _lstm_body,
        grid=(L // _TCH,),
        in_specs=[
            pl.BlockSpec((_TCH, B, G4), lambda i: (i, 0, 0)),
            pl.BlockSpec((_TCH, B, G4), lambda i: (i, 0, 0)),
            pl.BlockSpec((U, G4), lambda i: (0, 0)),
        ],
        out_specs=pl.BlockSpec((B, U), lambda i: (0, 0)),
        out_shape=jax.ShapeDtypeStruct((B, U), jnp.float32),
        scratch_shapes=[
            pltpu.VMEM((B, U), jnp.float32),
            pltpu.VMEM((B, U), jnp.float32),
            pltpu.VMEM((_TCH, B, G4), jnp.float32),
        ],
    )(g3, zx0, rec)


_BB = 32
_VCH = 2048
_NVC = (V + _VCH - 1) // _VCH
_TAIL = V - (_NVC - 1) * _VCH


def _head_body(h_ref, w_ref, b_ref, out_ref):
    hv = h_ref[...]
    total = jnp.zeros((_BB, 1), jnp.float32)
    for j in range(_NVC):
        lo = j * _VCH
        w = _VCH if j < _NVC - 1 else _TAIL
        lg = (jnp.dot(hv, w_ref[:, lo:lo + w],
                      preferred_element_type=jnp.float32)
              + b_ref[:, lo:lo + w])
        e = jnp.exp(lg)
        total = total + jnp.sum(e, axis=1, keepdims=True)
        out_ref[:, lo:lo + w] = e
    inv = 1.0 / total
    for j in range(_NVC):
        lo = j * _VCH
        w = _VCH if j < _NVC - 1 else _TAIL
        out_ref[:, lo:lo + w] = out_ref[:, lo:lo + w] * inv


def _softmax_head(h, dense_W, b2d):
    return pl.pallas_call(
        _head_body,
        grid=(B // _BB,),
        in_specs=[
            pl.BlockSpec((_BB, U), lambda i: (i, 0)),
            pl.BlockSpec((U, V), lambda i: (0, 0)),
            pl.BlockSpec((1, V), lambda i: (0, 0)),
        ],
        out_specs=pl.BlockSpec((_BB, V), lambda i: (i, 0)),
        out_shape=jax.ShapeDtypeStruct((B, V), jnp.float32),
    )(h, dense_W, b2d)


def kernel(x0, x1, emb_table, lstm_kernel, lstm_rec, lstm_bias, dense_W,
           dense_b):
    w_e = lstm_kernel[:EMB]
    w_f = lstm_kernel[EMB:]
    bias2d = lstm_bias.reshape(1, G4)

    proj = _project_table(emb_table, w_e)

    idx3d = x0.T.reshape(_NW, _ROWS_W // _JCH, _JCH)
    g = _sc_gather(proj, idx3d)
    g3 = g.reshape(L, B, G4)

    zx0 = _x1_proj(x1, w_f, bias2d)
    h = _lstm(g3, zx0, lstm_rec)

    return _softmax_head(h, dense_W, dense_b.reshape(1, V))


# speedup vs baseline: 1.0959x; 1.0959x over previous
import functools

import jax
import jax.numpy as jnp
from jax import lax
from jax.experimental import pallas as pl
from jax.experimental.pallas import tpu as pltpu
from jax.experimental.pallas import tpu_sc as plsc

V = 100000
EMB = 400
U = 16
G4 = 4 * U
GP = 128
FEAT = 16
B = 1024
L = 200
LB = B * L


_S1_ROWS = 1000


def _proj_body(emb_ref, we_ref, out_ref):
    out_ref[...] = jnp.dot(emb_ref[...], we_ref[...],
                           preferred_element_type=jnp.float32)


def _project_table(emb_table, w_e_pad):
    return pl.pallas_call(
        _proj_body,
        grid=(V // _S1_ROWS,),
        in_specs=[
            pl.BlockSpec((_S1_ROWS, EMB), lambda i: (i, 0)),
            pl.BlockSpec((EMB, GP), lambda i: (0, 0)),
        ],
        out_specs=pl.BlockSpec((_S1_ROWS, GP), lambda i: (i, 0)),
        out_shape=jax.ShapeDtypeStruct((V, GP), jnp.float32),
    )(emb_table, w_e_pad)


_NC, _NS = 2, 16
_NW = _NC * _NS
_ROWS_W = LB // _NW
_JCH = 128
_JPS = 5
_SUP = _ROWS_W // (_JPS * _JCH)


def _sc_gather_body(table_hbm, idx_hbm, out_hbm, idx_v, rows_v, sem):
    wid = lax.axis_index("s") * _NC + lax.axis_index("c")
    row0 = wid * _ROWS_W
    pltpu.sync_copy(idx_hbm.at[wid], idx_v)

    @pl.loop(0, _SUP)
    def _super(s):
        copies = []
        for j in range(_JPS):
            copies.append(pltpu.async_copy(
                table_hbm.at[idx_v.at[s * _JPS + j]],
                rows_v.at[pl.ds(j * _JCH, _JCH)], sem))
        for c in copies:
            c.wait()
        pltpu.sync_copy(
            rows_v, out_hbm.at[pl.ds(row0 + s * _JPS * _JCH, _JPS * _JCH)])


@functools.cache
def _make_sc_gather():
    return functools.partial(
        pl.kernel,
        out_type=jax.ShapeDtypeStruct((LB, GP), jnp.float32),
        mesh=plsc.VectorSubcoreMesh(core_axis_name="c", subcore_axis_name="s"),
        scratch_types=[
            pltpu.VMEM((_ROWS_W // _JCH, _JCH), jnp.int32),
            pltpu.VMEM((_JPS * _JCH, GP), jnp.float32),
            pltpu.SemaphoreType.DMA,
        ],
        compiler_params=pltpu.CompilerParams(use_tc_tiling_on_sc=False),
    )(_sc_gather_body)


def _sc_gather(table, idx3d):
    return _make_sc_gather()(table, idx3d)


_TCH = 8


def _lstm_body(g_ref, x1_ref, wf_ref, rec_ref, bias_ref, h_out_ref,
               h_s, c_s, zx_s):
    i = pl.program_id(0)

    @pl.when(i == 0)
    def _init():
        h_s[...] = jnp.zeros_like(h_s)
        c_s[...] = jnp.zeros_like(c_s)

    for tt in range(_TCH):
        zx_s[tt] = (g_ref[tt]
                    + jnp.dot(x1_ref[:, tt, :], wf_ref[...],
                              preferred_element_type=jnp.float32)
                    + bias_ref[...])

    def step(t, hc):
        h, c = hc
        z = zx_s[t] + jnp.dot(h, rec_ref[...],
                              preferred_element_type=jnp.float32)
        i_g = jax.nn.sigmoid(z[:, 0:U])
        f_g = jax.nn.sigmoid(z[:, U:2 * U])
        g_g = jnp.tanh(z[:, 2 * U:3 * U])
        o_g = jax.nn.sigmoid(z[:, 3 * U:4 * U])
        c_new = f_g * c + i_g * g_g
        h_new = o_g * jnp.tanh(c_new)
        return h_new, c_new

    h, c = lax.fori_loop(0, _TCH, step, (h_s[...], c_s[...]))
    h_s[...] = h
    c_s[...] = c
    h_out_ref[...] = h


def _lstm(g3, x1, w_f_pad, rec_pad, bias2d_pad):
    return pl.pallas_call(
        _lstm_body,
        grid=(L // _TCH,),
        in_specs=[
            pl.BlockSpec((_TCH, B, GP), lambda i: (i, 0, 0)),
            pl.BlockSpec((B, _TCH, FEAT), lambda i: (0, i, 0)),
            pl.BlockSpec((FEAT, GP), lambda i: (0, 0)),
            pl.BlockSpec((U, GP), lambda i: (0, 0)),
            pl.BlockSpec((1, GP), lambda i: (0, 0)),
        ],
        out_specs=pl.BlockSpec((B, U), lambda i: (0, 0)),
        out_shape=jax.ShapeDtypeStruct((B, U), jnp.float32),
        scratch_shapes=[
            pltpu.VMEM((B, U), jnp.float32),
            pltpu.VMEM((B, U), jnp.float32),
            pltpu.VMEM((_TCH, B, GP), jnp.float32),
        ],
    )(g3, x1, w_f_pad, rec_pad, bias2d_pad)


_BB = 32
_VCH = 2048
_NVC = (V + _VCH - 1) // _VCH
_TAIL = V - (_NVC - 1) * _VCH


def _head_body(h_ref, w_ref, b_ref, out_ref):
    hv = h_ref[...]
    total = jnp.zeros((_BB, 1), jnp.float32)
    for j in range(_NVC):
        lo = j * _VCH
        w = _VCH if j < _NVC - 1 else _TAIL
        lg = (jnp.dot(hv, w_ref[:, lo:lo + w],
                      preferred_element_type=jnp.float32)
              + b_ref[:, lo:lo + w])
        e = jnp.exp(lg)
        total = total + jnp.sum(e, axis=1, keepdims=True)
        out_ref[:, lo:lo + w] = e
    inv = 1.0 / total
    for j in range(_NVC):
        lo = j * _VCH
        w = _VCH if j < _NVC - 1 else _TAIL
        out_ref[:, lo:lo + w] = out_ref[:, lo:lo + w] * inv


def _softmax_head(h, dense_W, b2d):
    return pl.pallas_call(
        _head_body,
        grid=(B // _BB,),
        in_specs=[
            pl.BlockSpec((_BB, U), lambda i: (i, 0)),
            pl.BlockSpec((U, V), lambda i: (0, 0)),
            pl.BlockSpec((1, V), lambda i: (0, 0)),
        ],
        out_specs=pl.BlockSpec((_BB, V), lambda i: (i, 0)),
        out_shape=jax.ShapeDtypeStruct((B, V), jnp.float32),
    )(h, dense_W, b2d)


def kernel(x0, x1, emb_table, lstm_kernel, lstm_rec, lstm_bias, dense_W,
           dense_b):
    w_e_pad = jnp.pad(lstm_kernel[:EMB], ((0, 0), (0, GP - G4)))
    w_f_pad = jnp.pad(lstm_kernel[EMB:], ((0, 0), (0, GP - G4)))
    rec_pad = jnp.pad(lstm_rec, ((0, 0), (0, GP - G4)))
    bias2d_pad = jnp.pad(lstm_bias.reshape(1, G4), ((0, 0), (0, GP - G4)))

    proj = _project_table(emb_table, w_e_pad)

    idx3d = x0.T.reshape(_NW, _ROWS_W // _JCH, _JCH)
    g = _sc_gather(proj, idx3d)
    g3 = g.reshape(L, B, GP)

    h = _lstm(g3, x1, w_f_pad, rec_pad, bias2d_pad)

    return _softmax_head(h, dense_W, dense_b.reshape(1, V))


# speedup vs baseline: 2.4736x; 2.2572x over previous
import functools

import jax
import jax.numpy as jnp
from jax import lax
from jax.experimental import pallas as pl
from jax.experimental.pallas import tpu as pltpu
from jax.experimental.pallas import tpu_sc as plsc

V = 100000
EMB = 400
U = 16
G4 = 4 * U
GP = 128
HA = 24
FEAT = 16
B = 1024
L = 200
LB = B * L

_C00 = (((0,), (0,)), ((), ()))


_S1_ROWS = 1024


def _proj_body(embt_ref, we_ref, out_ref):
    out_ref[...] = lax.dot_general(embt_ref[...], we_ref[...], _C00,
                                   preferred_element_type=jnp.float32)


def _project_table(embT, w_e_pad):
    return pl.pallas_call(
        _proj_body,
        grid=(pl.cdiv(V, _S1_ROWS),),
        in_specs=[
            pl.BlockSpec((EMB, _S1_ROWS), lambda i: (0, i)),
            pl.BlockSpec((EMB, GP), lambda i: (0, 0)),
        ],
        out_specs=pl.BlockSpec((_S1_ROWS, GP), lambda i: (i, 0)),
        out_shape=jax.ShapeDtypeStruct((V, GP), jnp.float32),
    )(embT, w_e_pad)


_NC, _NS = 2, 16
_NW = _NC * _NS
_ROWS_W = LB // _NW
_JCH = 128
_JPS = 5
_SUP = _ROWS_W // (_JPS * _JCH)


def _sc_gather_body(table_hbm, idx_hbm, out_hbm, idx_v, rows_v, sem):
    wid = lax.axis_index("s") * _NC + lax.axis_index("c")
    row0 = wid * _ROWS_W
    pltpu.sync_copy(idx_hbm.at[wid], idx_v)

    @pl.loop(0, _SUP)
    def _super(s):
        copies = []
        for j in range(_JPS):
            copies.append(pltpu.async_copy(
                table_hbm.at[idx_v.at[s * _JPS + j]],
                rows_v.at[pl.ds(j * _JCH, _JCH)], sem))
        for c in copies:
            c.wait()
        pltpu.sync_copy(
            rows_v, out_hbm.at[pl.ds(row0 + s * _JPS * _JCH, _JPS * _JCH)])


@functools.cache
def _make_sc_gather():
    return functools.partial(
        pl.kernel,
        out_type=jax.ShapeDtypeStruct((LB, GP), jnp.float32),
        mesh=plsc.VectorSubcoreMesh(core_axis_name="c", subcore_axis_name="s"),
        scratch_types=[
            pltpu.VMEM((_ROWS_W // _JCH, _JCH), jnp.int32),
            pltpu.VMEM((_JPS * _JCH, GP), jnp.float32),
            pltpu.SemaphoreType.DMA,
        ],
        compiler_params=pltpu.CompilerParams(use_tc_tiling_on_sc=False),
    )(_sc_gather_body)


def _sc_gather(table, idx3d):
    return _make_sc_gather()(table, idx3d)


_TCH = 8


def _lstm_body(g_ref, x1_ref, wf_ref, rec_ref, biast_ref, h_out_ref,
               h_s, c_s, zx_s):
    i = pl.program_id(0)

    @pl.when(i == 0)
    def _init():
        h_s[...] = jnp.zeros_like(h_s)
        c_s[...] = jnp.zeros_like(c_s)

    for tt in range(_TCH):
        gT = jnp.swapaxes(g_ref[tt][:, 0:G4], 0, 1)
        zx_s[tt] = (gT
                    + lax.dot_general(wf_ref[...], x1_ref[tt], _C00,
                                      preferred_element_type=jnp.float32)
                    + biast_ref[...])

    def step(t, hc):
        hT, cT = hc
        zT = zx_s[t] + lax.dot_general(rec_ref[...], hT, _C00,
                                       preferred_element_type=jnp.float32)
        i_g = jax.nn.sigmoid(zT[0:U])
        f_g = jax.nn.sigmoid(zT[U:2 * U])
        g_g = jnp.tanh(zT[2 * U:3 * U])
        o_g = jax.nn.sigmoid(zT[3 * U:4 * U])
        c_new = f_g * cT + i_g * g_g
        h_new = o_g * jnp.tanh(c_new)
        return h_new, c_new

    hT, cT = lax.fori_loop(0, _TCH, step, (h_s[...], c_s[...]))
    h_s[...] = hT
    c_s[...] = cT
    h_out_ref[0:U] = hT
    row = lax.broadcasted_iota(jnp.int32, (HA - U, B), 0)
    h_out_ref[U:HA] = jnp.where(row == 0, 1.0, 0.0).astype(jnp.float32)


def _lstm(g3, x1t, w_f, rec, biasT):
    return pl.pallas_call(
        _lstm_body,
        grid=(L // _TCH,),
        in_specs=[
            pl.BlockSpec((_TCH, B, GP), lambda i: (i, 0, 0)),
            pl.BlockSpec((_TCH, FEAT, B), lambda i: (i, 0, 0)),
            pl.BlockSpec((FEAT, G4), lambda i: (0, 0)),
            pl.BlockSpec((U, G4), lambda i: (0, 0)),
            pl.BlockSpec((G4, 1), lambda i: (0, 0)),
        ],
        out_specs=pl.BlockSpec((HA, B), lambda i: (0, 0)),
        out_shape=jax.ShapeDtypeStruct((HA, B), jnp.float32),
        scratch_shapes=[
            pltpu.VMEM((U, B), jnp.float32),
            pltpu.VMEM((U, B), jnp.float32),
            pltpu.VMEM((_TCH, G4, B), jnp.float32),
        ],
    )(g3, x1t, w_f, rec, biasT)


_VB = 4096
_NVB = (V + _VB - 1) // _VB
_SUB = 256
_OFFS = list(range(0, _VB, _SUB))


def _den_body(w_ref, h_ref, inv_ref, acc_s):
    i = pl.program_id(0)

    @pl.when(i == 0)
    def _init():
        acc_s[...] = jnp.zeros_like(acc_s)

    hv = h_ref[...]
    part = jnp.zeros((1, B), jnp.float32)
    for off in _OFFS:
        lg = lax.dot_general(w_ref[:, off:off + _SUB], hv, _C00,
                             preferred_element_type=jnp.float32)
        e = jnp.exp(lg)
        gid = (i * _VB + off
               + lax.broadcasted_iota(jnp.int32, (_SUB, B), 0))
        e = jnp.where(gid < V, e, 0.0)
        part = part + jnp.sum(e, axis=0, keepdims=True)
    acc_s[...] = acc_s[...] + part

    @pl.when(i == _NVB - 1)
    def _fin():
        inv_ref[...] = 1.0 / acc_s[...]


def _head_denom(w_aug, h_aug):
    return pl.pallas_call(
        _den_body,
        grid=(_NVB,),
        in_specs=[
            pl.BlockSpec((HA, _VB), lambda i: (0, i)),
            pl.BlockSpec((HA, B), lambda i: (0, 0)),
        ],
        out_specs=pl.BlockSpec((1, B), lambda i: (0, 0)),
        out_shape=jax.ShapeDtypeStruct((1, B), jnp.float32),
        scratch_shapes=[pltpu.VMEM((1, B), jnp.float32)],
    )(w_aug, h_aug)


def _wr_body(w_ref, h_ref, inv_ref, out_ref):
    hv = h_ref[...]
    inv = inv_ref[...]
    for off in _OFFS:
        lg = lax.dot_general(w_ref[:, off:off + _SUB], hv, _C00,
                             preferred_element_type=jnp.float32)
        out_ref[pl.ds(off, _SUB), :] = jnp.exp(lg) * inv


def _head_write(w_aug, h_aug, inv):
    return pl.pallas_call(
        _wr_body,
        grid=(_NVB,),
        in_specs=[
            pl.BlockSpec((HA, _VB), lambda i: (0, i)),
            pl.BlockSpec((HA, B), lambda i: (0, 0)),
            pl.BlockSpec((1, B), lambda i: (0, 0)),
        ],
        out_specs=pl.BlockSpec((_VB, B), lambda i: (i, 0)),
        out_shape=jax.ShapeDtypeStruct((V, B), jnp.float32),
    )(w_aug, h_aug, inv)


def kernel(x0, x1, emb_table, lstm_kernel, lstm_rec, lstm_bias, dense_W,
           dense_b):
    embT = jnp.swapaxes(emb_table, 0, 1)
    w_e_pad = jnp.pad(lstm_kernel[:EMB], ((0, 0), (0, GP - G4)))
    w_f = lstm_kernel[EMB:]
    biasT = lstm_bias.reshape(G4, 1)

    proj = _project_table(embT, w_e_pad)

    idx3d = x0.T.reshape(_NW, _ROWS_W // _JCH, _JCH)
    g = _sc_gather(proj, idx3d)
    g3 = g.reshape(L, B, GP)

    x1t = jnp.transpose(x1, (1, 2, 0))
    h_aug = _lstm(g3, x1t, w_f, lstm_rec, biasT)

    w_aug = jnp.concatenate(
        [dense_W, dense_b.reshape(1, V),
         jnp.zeros((HA - U - 1, V), jnp.float32)], axis=0)
    inv = _head_denom(w_aug, h_aug)
    outT = _head_write(w_aug, h_aug, inv)
    return jnp.swapaxes(outT, 0, 1)


# speedup vs baseline: 2.6150x; 1.0572x over previous
import functools

import jax
import jax.numpy as jnp
from jax import lax
from jax.experimental import pallas as pl
from jax.experimental.pallas import tpu as pltpu
from jax.experimental.pallas import tpu_sc as plsc

V = 100000
EMB = 400
U = 16
G4 = 4 * U
GP = 128
HA = 24
FEAT = 16
B = 1024
L = 200
LB = B * L

_C00 = (((0,), (0,)), ((), ()))


_S1_ROWS = 2048


def _proj_body(embt_ref, we_ref, out_ref):
    out_ref[...] = lax.dot_general(embt_ref[...], we_ref[...], _C00,
                                   preferred_element_type=jnp.float32)


def _project_table(embT, w_e_pad):
    return pl.pallas_call(
        _proj_body,
        grid=(pl.cdiv(V, _S1_ROWS),),
        in_specs=[
            pl.BlockSpec((EMB, _S1_ROWS), lambda i: (0, i)),
            pl.BlockSpec((EMB, GP), lambda i: (0, 0)),
        ],
        out_specs=pl.BlockSpec((_S1_ROWS, GP), lambda i: (i, 0)),
        out_shape=jax.ShapeDtypeStruct((V, GP), jnp.float32),
    )(embT, w_e_pad)


_NC, _NS = 2, 16
_NW = _NC * _NS
_ROWS_W = LB // _NW
_JCH = 128
_JPS = 5
_SUP = _ROWS_W // (_JPS * _JCH)


def _sc_gather_body(table_hbm, idx_hbm, out_hbm, idx_v, rows_v, sem):
    wid = lax.axis_index("s") * _NC + lax.axis_index("c")
    row0 = wid * _ROWS_W
    pltpu.sync_copy(idx_hbm.at[wid], idx_v)

    @pl.loop(0, _SUP)
    def _super(s):
        copies = []
        for j in range(_JPS):
            copies.append(pltpu.async_copy(
                table_hbm.at[idx_v.at[s * _JPS + j]],
                rows_v.at[pl.ds(j * _JCH, _JCH)], sem))
        for c in copies:
            c.wait()
        pltpu.sync_copy(
            rows_v, out_hbm.at[pl.ds(row0 + s * _JPS * _JCH, _JPS * _JCH)])


@functools.cache
def _make_sc_gather():
    return functools.partial(
        pl.kernel,
        out_type=jax.ShapeDtypeStruct((LB, GP), jnp.float32),
        mesh=plsc.VectorSubcoreMesh(core_axis_name="c", subcore_axis_name="s"),
        scratch_types=[
            pltpu.VMEM((_ROWS_W // _JCH, _JCH), jnp.int32),
            pltpu.VMEM((_JPS * _JCH, GP), jnp.float32),
            pltpu.SemaphoreType.DMA,
        ],
        compiler_params=pltpu.CompilerParams(use_tc_tiling_on_sc=False),
    )(_sc_gather_body)


def _sc_gather(table, idx3d):
    return _make_sc_gather()(table, idx3d)


_TCH = 20


def _lstm_body(g_ref, x1_ref, wf_ref, rec_ref, biast_ref, h_out_ref,
               h_s, c_s, zx_s):
    i = pl.program_id(0)

    @pl.when(i == 0)
    def _init():
        h_s[...] = jnp.zeros_like(h_s)
        c_s[...] = jnp.zeros_like(c_s)

    for tt in range(_TCH):
        gT = jnp.swapaxes(g_ref[tt][:, 0:G4], 0, 1)
        zx_s[tt] = (gT
                    + lax.dot_general(wf_ref[...], x1_ref[tt], _C00,
                                      preferred_element_type=jnp.float32)
                    + biast_ref[...])

    def step(t, hc):
        hT, cT = hc
        zT = zx_s[t] + lax.dot_general(rec_ref[...], hT, _C00,
                                       preferred_element_type=jnp.float32)
        i_g = jax.nn.sigmoid(zT[0:U])
        f_g = jax.nn.sigmoid(zT[U:2 * U])
        g_g = jnp.tanh(zT[2 * U:3 * U])
        o_g = jax.nn.sigmoid(zT[3 * U:4 * U])
        c_new = f_g * cT + i_g * g_g
        h_new = o_g * jnp.tanh(c_new)
        return h_new, c_new

    hT, cT = lax.fori_loop(0, _TCH, step, (h_s[...], c_s[...]))
    h_s[...] = hT
    c_s[...] = cT
    h_out_ref[0:U] = hT
    row = lax.broadcasted_iota(jnp.int32, (HA - U, B), 0)
    h_out_ref[U:HA] = jnp.where(row == 0, 1.0, 0.0).astype(jnp.float32)


def _lstm(g3, x1t, w_f, rec, biasT):
    return pl.pallas_call(
        _lstm_body,
        grid=(L // _TCH,),
        in_specs=[
            pl.BlockSpec((_TCH, B, GP), lambda i: (i, 0, 0)),
            pl.BlockSpec((_TCH, FEAT, B), lambda i: (i, 0, 0)),
            pl.BlockSpec((FEAT, G4), lambda i: (0, 0)),
            pl.BlockSpec((U, G4), lambda i: (0, 0)),
            pl.BlockSpec((G4, 1), lambda i: (0, 0)),
        ],
        out_specs=pl.BlockSpec((HA, B), lambda i: (0, 0)),
        out_shape=jax.ShapeDtypeStruct((HA, B), jnp.float32),
        scratch_shapes=[
            pltpu.VMEM((U, B), jnp.float32),
            pltpu.VMEM((U, B), jnp.float32),
            pltpu.VMEM((_TCH, G4, B), jnp.float32),
        ],
    )(g3, x1t, w_f, rec, biasT)


_VB = 4096
_NVB = (V + _VB - 1) // _VB
_SUB = 256
_OFFS = list(range(0, _VB, _SUB))


def _den_body(w_ref, h_ref, inv_ref, acc_s):
    i = pl.program_id(0)

    @pl.when(i == 0)
    def _init():
        acc_s[...] = jnp.zeros_like(acc_s)

    hv = h_ref[...]
    part = jnp.zeros((1, B), jnp.float32)
    for off in _OFFS:
        lg = lax.dot_general(w_ref[:, off:off + _SUB], hv, _C00,
                             preferred_element_type=jnp.float32)
        e = jnp.exp(lg)
        gid = (i * _VB + off
               + lax.broadcasted_iota(jnp.int32, (_SUB, B), 0))
        e = jnp.where(gid < V, e, 0.0)
        part = part + jnp.sum(e, axis=0, keepdims=True)
    acc_s[...] = acc_s[...] + part

    @pl.when(i == _NVB - 1)
    def _fin():
        inv_ref[...] = 1.0 / acc_s[...]


def _head_denom(w_aug, h_aug):
    return pl.pallas_call(
        _den_body,
        grid=(_NVB,),
        in_specs=[
            pl.BlockSpec((HA, _VB), lambda i: (0, i)),
            pl.BlockSpec((HA, B), lambda i: (0, 0)),
        ],
        out_specs=pl.BlockSpec((1, B), lambda i: (0, 0)),
        out_shape=jax.ShapeDtypeStruct((1, B), jnp.float32),
        scratch_shapes=[pltpu.VMEM((1, B), jnp.float32)],
    )(w_aug, h_aug)


def _wr_body(w_ref, h_ref, inv_ref, out_ref):
    hv = h_ref[...]
    inv = inv_ref[...]
    for off in _OFFS:
        lg = lax.dot_general(w_ref[:, off:off + _SUB], hv, _C00,
                             preferred_element_type=jnp.float32)
        out_ref[pl.ds(off, _SUB), :] = jnp.exp(lg) * inv


def _head_write(w_aug, h_aug, inv):
    return pl.pallas_call(
        _wr_body,
        grid=(_NVB,),
        in_specs=[
            pl.BlockSpec((HA, _VB), lambda i: (0, i)),
            pl.BlockSpec((HA, B), lambda i: (0, 0)),
            pl.BlockSpec((1, B), lambda i: (0, 0)),
        ],
        out_specs=pl.BlockSpec((_VB, B), lambda i: (i, 0)),
        out_shape=jax.ShapeDtypeStruct((V, B), jnp.float32),
    )(w_aug, h_aug, inv)


def kernel(x0, x1, emb_table, lstm_kernel, lstm_rec, lstm_bias, dense_W,
           dense_b):
    embT = jnp.swapaxes(emb_table, 0, 1)
    w_e_pad = jnp.pad(lstm_kernel[:EMB], ((0, 0), (0, GP - G4)))
    w_f = lstm_kernel[EMB:]
    biasT = lstm_bias.reshape(G4, 1)

    proj = _project_table(embT, w_e_pad)

    idx3d = x0.T.reshape(_NW, _ROWS_W // _JCH, _JCH)
    g = _sc_gather(proj, idx3d)
    g3 = g.reshape(L, B, GP)

    x1t = jnp.transpose(x1, (1, 2, 0))
    h_aug = _lstm(g3, x1t, w_f, lstm_rec, biasT)

    w_aug = jnp.concatenate(
        [dense_W, dense_b.reshape(1, V),
         jnp.zeros((HA - U - 1, V), jnp.float32)], axis=0)
    w_aug = w_aug.astype(jnp.bfloat16)
    h_bf = h_aug.astype(jnp.bfloat16)
    inv = _head_denom(w_aug, h_bf)
    outT = _head_write(w_aug, h_bf, inv)
    return jnp.swapaxes(outT, 0, 1)


# speedup vs baseline: 2.6486x; 1.0129x over previous
import functools

import jax
import jax.numpy as jnp
from jax import lax
from jax.experimental import pallas as pl
from jax.experimental.pallas import tpu as pltpu
from jax.experimental.pallas import tpu_sc as plsc

V = 100000
EMB = 400
U = 16
G4 = 4 * U
GP = 128
HA = 24
FEAT = 16
B = 1024
L = 200
LB = B * L

_C00 = (((0,), (0,)), ((), ()))


_S1_ROWS = 2048


def _proj_body(embt_ref, we_ref, out_ref):
    out_ref[...] = lax.dot_general(embt_ref[...], we_ref[...], _C00,
                                   preferred_element_type=jnp.float32)


def _project_table(embT, w_e_pad):
    return pl.pallas_call(
        _proj_body,
        grid=(pl.cdiv(V, _S1_ROWS),),
        in_specs=[
            pl.BlockSpec((EMB, _S1_ROWS), lambda i: (0, i)),
            pl.BlockSpec((EMB, GP), lambda i: (0, 0)),
        ],
        out_specs=pl.BlockSpec((_S1_ROWS, GP), lambda i: (i, 0)),
        out_shape=jax.ShapeDtypeStruct((V, GP), jnp.float32),
    )(embT, w_e_pad)


_NC, _NS = 2, 16
_NW = _NC * _NS
_ROWS_W = LB // _NW
_JCH = 128
_JPS = 2
_SCH = _JPS * _JCH
_SUP = _ROWS_W // _SCH


def _sc_gather_body(table_hbm, idx_hbm, out_hbm, idx_v, rows_v, gsem, wsem):
    wid = lax.axis_index("s") * _NC + lax.axis_index("c")
    row0 = wid * _ROWS_W
    pltpu.sync_copy(idx_hbm.at[wid], idx_v)

    def fire(s, b):
        for j in range(_JPS):
            pltpu.async_copy(
                table_hbm.at[idx_v.at[s * _JPS + j]],
                rows_v.at[b, pl.ds(j * _JCH, _JCH)], gsem)

    def drain_gathers(b):
        for j in range(_JPS):
            pltpu.make_async_copy(
                table_hbm.at[pl.ds(0, _JCH)],
                rows_v.at[b, pl.ds(j * _JCH, _JCH)], gsem).wait()

    def drain_write(b):
        pltpu.make_async_copy(
            rows_v.at[b], out_hbm.at[pl.ds(row0, _SCH)], wsem).wait()

    fire(0, 0)

    @pl.loop(0, _SUP)
    def _super(s):
        b = s % 2

        @pl.when(s + 1 < _SUP)
        def _prefetch():
            @pl.when(s >= 1)
            def _reuse():
                drain_write(1 - b)
            fire(s + 1, 1 - b)

        drain_gathers(b)
        pltpu.async_copy(
            rows_v.at[b], out_hbm.at[pl.ds(row0 + s * _SCH, _SCH)], wsem)

    drain_write(_SUP % 2)
    drain_write(1 - _SUP % 2)


@functools.cache
def _make_sc_gather():
    return functools.partial(
        pl.kernel,
        out_type=jax.ShapeDtypeStruct((LB, GP), jnp.float32),
        mesh=plsc.VectorSubcoreMesh(core_axis_name="c", subcore_axis_name="s"),
        scratch_types=[
            pltpu.VMEM((_ROWS_W // _JCH, _JCH), jnp.int32),
            pltpu.VMEM((2, _SCH, GP), jnp.float32),
            pltpu.SemaphoreType.DMA,
            pltpu.SemaphoreType.DMA,
        ],
        compiler_params=pltpu.CompilerParams(use_tc_tiling_on_sc=False),
    )(_sc_gather_body)


def _sc_gather(table, idx3d):
    return _make_sc_gather()(table, idx3d)


_TCH = 20


def _lstm_body(g_ref, x1_ref, wf_ref, rec_ref, biast_ref, h_out_ref,
               h_s, c_s, zx_s):
    i = pl.program_id(0)

    @pl.when(i == 0)
    def _init():
        h_s[...] = jnp.zeros_like(h_s)
        c_s[...] = jnp.zeros_like(c_s)

    for tt in range(_TCH):
        gT = jnp.swapaxes(g_ref[tt][:, 0:G4], 0, 1)
        zx_s[tt] = (gT
                    + lax.dot_general(wf_ref[...], x1_ref[tt], _C00,
                                      preferred_element_type=jnp.float32)
                    + biast_ref[...])

    def step(t, hc):
        hT, cT = hc
        zT = zx_s[t] + lax.dot_general(rec_ref[...], hT, _C00,
                                       preferred_element_type=jnp.float32)
        i_g = jax.nn.sigmoid(zT[0:U])
        f_g = jax.nn.sigmoid(zT[U:2 * U])
        g_g = jnp.tanh(zT[2 * U:3 * U])
        o_g = jax.nn.sigmoid(zT[3 * U:4 * U])
        c_new = f_g * cT + i_g * g_g
        h_new = o_g * jnp.tanh(c_new)
        return h_new, c_new

    hT, cT = lax.fori_loop(0, _TCH, step, (h_s[...], c_s[...]))
    h_s[...] = hT
    c_s[...] = cT
    h_out_ref[0:U] = hT
    row = lax.broadcasted_iota(jnp.int32, (HA - U, B), 0)
    h_out_ref[U:HA] = jnp.where(row == 0, 1.0, 0.0).astype(jnp.float32)


def _lstm(g3, x1t, w_f, rec, biasT):
    return pl.pallas_call(
        _lstm_body,
        grid=(L // _TCH,),
        in_specs=[
            pl.BlockSpec((_TCH, B, GP), lambda i: (i, 0, 0)),
            pl.BlockSpec((_TCH, FEAT, B), lambda i: (i, 0, 0)),
            pl.BlockSpec((FEAT, G4), lambda i: (0, 0)),
            pl.BlockSpec((U, G4), lambda i: (0, 0)),
            pl.BlockSpec((G4, 1), lambda i: (0, 0)),
        ],
        out_specs=pl.BlockSpec((HA, B), lambda i: (0, 0)),
        out_shape=jax.ShapeDtypeStruct((HA, B), jnp.float32),
        scratch_shapes=[
            pltpu.VMEM((U, B), jnp.float32),
            pltpu.VMEM((U, B), jnp.float32),
            pltpu.VMEM((_TCH, G4, B), jnp.float32),
        ],
    )(g3, x1t, w_f, rec, biasT)


_VB = 4096
_NVB = (V + _VB - 1) // _VB
_SUB = 256
_OFFS = list(range(0, _VB, _SUB))


def _den_body(w_ref, h_ref, inv_ref, acc_s):
    i = pl.program_id(0)

    @pl.when(i == 0)
    def _init():
        acc_s[...] = jnp.zeros_like(acc_s)

    hv = h_ref[...]
    part = jnp.zeros((1, B), jnp.float32)
    for off in _OFFS:
        lg = lax.dot_general(w_ref[:, off:off + _SUB], hv, _C00,
                             preferred_element_type=jnp.float32)
        e = jnp.exp(lg)
        gid = (i * _VB + off
               + lax.broadcasted_iota(jnp.int32, (_SUB, B), 0))
        e = jnp.where(gid < V, e, 0.0)
        part = part + jnp.sum(e, axis=0, keepdims=True)
    acc_s[...] = acc_s[...] + part

    @pl.when(i == _NVB - 1)
    def _fin():
        inv_ref[...] = 1.0 / acc_s[...]


def _head_denom(w_aug, h_aug):
    return pl.pallas_call(
        _den_body,
        grid=(_NVB,),
        in_specs=[
            pl.BlockSpec((HA, _VB), lambda i: (0, i)),
            pl.BlockSpec((HA, B), lambda i: (0, 0)),
        ],
        out_specs=pl.BlockSpec((1, B), lambda i: (0, 0)),
        out_shape=jax.ShapeDtypeStruct((1, B), jnp.float32),
        scratch_shapes=[pltpu.VMEM((1, B), jnp.float32)],
    )(w_aug, h_aug)


def _wr_body(w_ref, h_ref, inv_ref, out_ref):
    hv = h_ref[...]
    inv = inv_ref[...]
    for off in _OFFS:
        lg = lax.dot_general(w_ref[:, off:off + _SUB], hv, _C00,
                             preferred_element_type=jnp.float32)
        out_ref[pl.ds(off, _SUB), :] = jnp.exp(lg) * inv


def _head_write(w_aug, h_aug, inv):
    return pl.pallas_call(
        _wr_body,
        grid=(_NVB,),
        in_specs=[
            pl.BlockSpec((HA, _VB), lambda i: (0, i)),
            pl.BlockSpec((HA, B), lambda i: (0, 0)),
            pl.BlockSpec((1, B), lambda i: (0, 0)),
        ],
        out_specs=pl.BlockSpec((_VB, B), lambda i: (i, 0)),
        out_shape=jax.ShapeDtypeStruct((V, B), jnp.float32),
    )(w_aug, h_aug, inv)


def kernel(x0, x1, emb_table, lstm_kernel, lstm_rec, lstm_bias, dense_W,
           dense_b):
    embT = jnp.swapaxes(emb_table, 0, 1)
    w_e_pad = jnp.pad(lstm_kernel[:EMB], ((0, 0), (0, GP - G4)))
    w_f = lstm_kernel[EMB:]
    biasT = lstm_bias.reshape(G4, 1)

    proj = _project_table(embT, w_e_pad)

    idx3d = x0.T.reshape(_NW, _ROWS_W // _JCH, _JCH)
    g = _sc_gather(proj, idx3d)
    g3 = g.reshape(L, B, GP)

    x1t = jnp.transpose(x1, (1, 2, 0))
    h_aug = _lstm(g3, x1t, w_f, lstm_rec, biasT)

    w_aug = jnp.concatenate(
        [dense_W, dense_b.reshape(1, V),
         jnp.zeros((HA - U - 1, V), jnp.float32)], axis=0)
    w_aug = w_aug.astype(jnp.bfloat16)
    h_bf = h_aug.astype(jnp.bfloat16)
    inv = _head_denom(w_aug, h_bf)
    outT = _head_write(w_aug, h_bf, inv)
    return jnp.swapaxes(outT, 0, 1)
